# baseline (speedup 1.0000x reference)
import functools

import jax
import jax.numpy as jnp
from jax import lax
from jax.experimental import pallas as pl
from jax.experimental.pallas import tpu as pltpu
from jax.experimental.pallas import tpu_sc as plsc

B = 16384
L = 25
H = 64

NC = 2
NS = 16
NW = NC * NS
CPW = B // NW
PPW = CPW * L
CCH = 16
PCH = CCH * L
NCH = CPW // CCH
NBUF = 2
SLICES = ((0, 128), (128, 128), (256, 128), (384, 16))

_mesh = plsc.VectorSubcoreMesh(core_axis_name="c", subcore_axis_name="s")


@functools.partial(
    pl.kernel,
    out_type=jax.ShapeDtypeStruct((B * L,), jnp.float32),
    mesh=_mesh,
    compiler_params=pltpu.CompilerParams(
        needs_layout_passes=False, use_tc_tiling_on_sc=False),
    scratch_types=[
        pltpu.VMEM((CPW,), jnp.int32),
        pltpu.VMEM((PPW,), jnp.int32),
        pltpu.VMEM((CPW, H), jnp.float32),
        pltpu.VMEM((PCH, H), jnp.float32),
        pltpu.VMEM((PCH, H), jnp.float32),
        pltpu.VMEM((PPW,), jnp.float32),
        pltpu.SemaphoreType.DMA,
        pltpu.SemaphoreType.DMA,
        pltpu.SemaphoreType.DMA,
    ],
)
def _skipgram_sc(cen_hbm, ctx_hbm, u_hbm, v_hbm, out_hbm,
                 cen_v, ctx_v, u_v, vb0, vb1, out_v,
                 usem, vsem0, vsem1):
    wid = lax.axis_index("s") * NC + lax.axis_index("c")
    base_c = pl.multiple_of(wid * CPW, CPW)
    base_p = pl.multiple_of(wid * PPW, PPW)

    pltpu.sync_copy(cen_hbm.at[pl.ds(base_c, CPW)], cen_v)
    pltpu.sync_copy(ctx_hbm.at[pl.ds(base_p, PPW)], ctx_v)

    u_copies = [
        pltpu.async_copy(
            u_hbm.at[cen_v.at[pl.ds(j * 128, 128)]],
            u_v.at[pl.ds(j * 128, 128)],
            usem,
        )
        for j in range(CPW // 128)
    ]

    vbufs = (vb0, vb1)
    vsems = (vsem0, vsem1)

    def fire(ch, b):
        for off, sz in SLICES:
            pltpu.async_copy(
                v_hbm.at[ctx_v.at[pl.ds(pl.multiple_of(ch * PCH + off, 8), sz)]],
                vbufs[b].at[pl.ds(off, sz)],
                vsems[b],
            )

    def drain(b):
        pltpu.make_async_copy(v_hbm.at[pl.ds(0, PCH)], vbufs[b], vsems[b]).wait()

    for b in range(NBUF):
        fire(b, b)
    for cp in u_copies:
        cp.wait()

    lanes = lax.iota(jnp.int32, 16)
    mask15 = lanes == 15

    def compute(ch, b):
        vb = vbufs[b]

        def center_body(i, carry):
            c = ch * CCH + i
            us = [u_v[c, pl.ds(k * 16, 16)] for k in range(H // 16)]
            for l in range(L):
                r = i * L + l
                acc = vb[r, pl.ds(0, 16)] * us[0]
                for k in range(1, H // 16):
                    acc = acc + vb[r, pl.ds(k * 16, 16)] * us[k]
                cum = plsc.cumsum(acc)
                p = ch * PCH + r
                plsc.store_scatter(
                    out_v, [jnp.full((16,), p, jnp.int32)], cum, mask=mask15)
            return carry

        lax.fori_loop(0, CCH, center_body, 0)

    def group(g, carry):
        for b in range(NBUF):
            ch = g * NBUF + b
            drain(b)
            compute(ch, b)

            @pl.when(ch + NBUF < NCH)
            def _():
                fire(ch + NBUF, b)
        return carry

    lax.fori_loop(0, NCH // NBUF, group, 0)

    pltpu.sync_copy(out_v, out_hbm.at[pl.ds(base_p, PPW)])


def kernel(center_ids, context_neg_ids, U, V):
    cen = center_ids.reshape(-1).astype(jnp.int32)
    ctx = context_neg_ids.reshape(-1).astype(jnp.int32)
    out = _skipgram_sc(cen, ctx, U, V)
    return out.reshape(B, L)


# speedup vs baseline: 1.1904x; 1.1904x over previous
import functools

import jax
import jax.numpy as jnp
from jax import lax
from jax.experimental import pallas as pl
from jax.experimental.pallas import tpu as pltpu
from jax.experimental.pallas import tpu_sc as plsc

B = 16384
L = 25
H = 64

NC = 2
NS = 16
NW = NC * NS
CPW = B // NW
PPW = CPW * L
CCH = 8
PCH = CCH * L
NCH = CPW // CCH
NBUF = 2

_mesh = plsc.VectorSubcoreMesh(core_axis_name="c", subcore_axis_name="s")


@functools.partial(
    pl.kernel,
    out_type=jax.ShapeDtypeStruct((B * L,), jnp.float32),
    mesh=_mesh,
    compiler_params=pltpu.CompilerParams(needs_layout_passes=False),
    scratch_types=[
        pltpu.VMEM((CPW + 16,), jnp.int32),
        pltpu.VMEM((PPW + 16,), jnp.int32),
        pltpu.VMEM((CCH, H), jnp.float32),
        pltpu.VMEM((CCH, H), jnp.float32),
        pltpu.VMEM((PCH, H), jnp.float32),
        pltpu.VMEM((PCH, H), jnp.float32),
        pltpu.VMEM((PPW,), jnp.float32),
        pltpu.SemaphoreType.DMA,
        pltpu.SemaphoreType.DMA,
        pltpu.SemaphoreType.DMA,
        pltpu.SemaphoreType.DMA,
    ],
)
def _skipgram_sc(cen_hbm, ctx_hbm, u_hbm, v_hbm, out_hbm,
                 cen_v, ctx_v, ub0, ub1, vb0, vb1, out_v,
                 usem0, usem1, vsem0, vsem1):
    wid = lax.axis_index("s") * NC + lax.axis_index("c")
    base_c = pl.multiple_of(wid * CPW, CPW)
    base_p = pl.multiple_of(wid * PPW, PPW)

    pltpu.sync_copy(cen_hbm.at[pl.ds(base_c, CPW)], cen_v.at[pl.ds(0, CPW)])
    pltpu.sync_copy(ctx_hbm.at[pl.ds(base_p, PPW)], ctx_v.at[pl.ds(0, PPW)])

    ubufs = (ub0, ub1)
    vbufs = (vb0, vb1)
    usems = (usem0, usem1)
    vsems = (vsem0, vsem1)

    def fire(ch, b):
        def u_body(i, carry):
            row = cen_v[pl.ds(ch * CCH + i, 16)][0]
            pltpu.async_copy(
                u_hbm.at[pl.ds(row, 1)], ubufs[b].at[pl.ds(i, 1)], usems[b])
            return carry

        lax.fori_loop(0, CCH, u_body, 0)

        def v_body(i, carry):
            row = ctx_v[pl.ds(ch * PCH + i, 16)][0]
            pltpu.async_copy(
                v_hbm.at[pl.ds(row, 1)], vbufs[b].at[pl.ds(i, 1)], vsems[b])
            return carry

        lax.fori_loop(0, PCH, v_body, 0)

    def drain(b):
        pltpu.make_async_copy(u_hbm.at[pl.ds(0, CCH)], ubufs[b], usems[b]).wait()
        pltpu.make_async_copy(v_hbm.at[pl.ds(0, PCH)], vbufs[b], vsems[b]).wait()

    for b in range(NBUF):
        fire(b, b)

    lanes = lax.iota(jnp.int32, 16)
    mask15 = lanes == 15

    def compute(ch, b):
        ub = ubufs[b]
        vb = vbufs[b]

        def center_body(i, carry):
            us = [ub[i, pl.ds(k * 16, 16)] for k in range(H // 16)]
            for l in range(L):
                r = i * L + l
                acc = vb[r, pl.ds(0, 16)] * us[0]
                for k in range(1, H // 16):
                    acc = acc + vb[r, pl.ds(k * 16, 16)] * us[k]
                cum = plsc.cumsum(acc)
                p = ch * PCH + r
                plsc.store_scatter(
                    out_v, [jnp.full((16,), p, jnp.int32)], cum, mask=mask15)
            return carry

        lax.fori_loop(0, CCH, center_body, 0)

    def group(g, carry):
        for b in range(NBUF):
            ch = g * NBUF + b
            drain(b)
            compute(ch, b)

            @pl.when(ch + NBUF < NCH)
            def _():
                fire(ch + NBUF, b)
        return carry

    lax.fori_loop(0, NCH // NBUF, group, 0)

    pltpu.sync_copy(out_v, out_hbm.at[pl.ds(base_p, PPW)])


def kernel(center_ids, context_neg_ids, U, V):
    cen = center_ids.reshape(-1).astype(jnp.int32)
    ctx = context_neg_ids.reshape(-1).astype(jnp.int32)
    out = _skipgram_sc(cen, ctx, U, V)
    return out.reshape(B, L)
